# baseline (speedup 1.0000x reference)
import functools

import jax
import jax.numpy as jnp
from jax import lax
from jax.experimental import pallas as pl
from jax.experimental.pallas import tpu as pltpu
from jax.experimental.pallas import tpu_sc as plsc

NNZ = 1638400
VOCAB_SIZE = 1000000
KDIM = 16
NBATCH = 16384

NC = 2
NS = 16
NW = NC * NS
PER_W = NNZ // NW
CHUNK = 1024
NCHUNK = PER_W // CHUNK
GB = 128
NGB = CHUNK // GB
ROWS_PER_TILE = NBATCH // NS


def _sc_body(vals_hbm, feat2d_hbm, rids2d_hbm, weight_hbm, out_hbm,
             idx_v, rids_v, vals_v, rows_v, wbuf_v, zbuf_v,
             acc_sh, sem):
    c_id = lax.axis_index("c")
    s_id = lax.axis_index("s")
    wid = c_id * NS + s_id
    base = wid * PER_W

    z16 = jnp.zeros((16,), jnp.float32)
    z16i = jnp.zeros((16,), jnp.int32)
    iota16 = lax.broadcasted_iota(jnp.int32, (16,), 0)
    iota16h = iota16 + 16

    def zero_zbuf(r, carry):
        plsc.store_scatter(zbuf_v, [z16i + r, iota16], z16)
        plsc.store_scatter(zbuf_v, [z16i + r, iota16h], z16)
        return carry

    lax.fori_loop(0, 128, zero_zbuf, 0)

    def zero_acc(t, carry):
        pltpu.sync_copy(zbuf_v, acc_sh.at[pl.ds(pl.multiple_of(s_id * ROWS_PER_TILE + t * 128, 128), 128)])
        return carry

    lax.fori_loop(0, ROWS_PER_TILE // 128, zero_acc, 0)
    plsc.subcore_barrier()

    def chunk_body(ci, carry):
        cb = pl.multiple_of(base + ci * CHUNK, CHUNK)
        row0 = pl.multiple_of(base // GB + ci * NGB, 8)
        pltpu.sync_copy(feat2d_hbm.at[pl.ds(row0, NGB)], idx_v)
        pltpu.sync_copy(rids2d_hbm.at[pl.ds(row0, NGB)], rids_v)
        pltpu.sync_copy(vals_hbm.at[pl.ds(cb, CHUNK)], vals_v)
        descs = [
            pltpu.async_copy(weight_hbm.at[idx_v.at[j]],
                             rows_v.at[pl.ds(j * GB, GB)], sem)
            for j in range(NGB)
        ]
        for d in descs:
            d.wait()

        def group_body(g, carry2):
            gb = g * 16
            v16 = vals_v[pl.ds(gb, 16)]
            ridx = z16i + gb
            for l in range(16):
                row = plsc.load_gather(rows_v, [ridx + l, iota16])
                w = row * v16[l]
                plsc.store_scatter(wbuf_v, [ridx + l, iota16], w)
                plsc.store_scatter(wbuf_v, [ridx + l, iota16h], w * w)
            return carry2

        lax.fori_loop(0, CHUNK // 16, group_body, 0)

        def scat(j, c2):
            pltpu.sync_copy(wbuf_v.at[pl.ds(j * GB, GB)],
                            acc_sh.at[rids_v.at[j]], add=True)
            return c2

        lax.fori_loop(0, NGB, scat, 0)
        return carry

    lax.fori_loop(0, NCHUNK, chunk_body, 0)
    plsc.subcore_barrier()
    out_base = pl.multiple_of(s_id * ROWS_PER_TILE, ROWS_PER_TILE)
    pltpu.sync_copy(acc_sh.at[pl.ds(out_base, ROWS_PER_TILE)],
                    out_hbm.at[c_id, pl.ds(out_base, ROWS_PER_TILE)])


_sc_kernel = functools.partial(
    pl.kernel,
    mesh=plsc.VectorSubcoreMesh(core_axis_name="c", subcore_axis_name="s",
                                num_cores=NC, num_subcores=NS),
    out_type=jax.ShapeDtypeStruct((NC, NBATCH, 32), jnp.float32),
    scratch_types=[
        pltpu.VMEM((NGB, GB), jnp.int32),
        pltpu.VMEM((NGB, GB), jnp.int32),
        pltpu.VMEM((CHUNK,), jnp.float32),
        pltpu.VMEM((CHUNK, KDIM), jnp.float32),
        pltpu.VMEM((CHUNK, 32), jnp.float32),
        pltpu.VMEM((128, 32), jnp.float32),
        pltpu.VMEM_SHARED((NBATCH, 32), jnp.float32),
        pltpu.SemaphoreType.DMA,
    ],
    compiler_params=pltpu.CompilerParams(needs_layout_passes=False, use_tc_tiling_on_sc=False),
)(_sc_body)


def _combine_body(p_ref, o_ref):
    x = p_ref[...]
    p = x[0] + x[1]
    k = lax.broadcasted_iota(jnp.int32, (NBATCH, 32), 1)
    t = jnp.where(k < KDIM, p * p, -p)
    o_ref[...] = jnp.sum(t, axis=1, keepdims=True)


_combine = pl.pallas_call(
    _combine_body,
    out_shape=jax.ShapeDtypeStruct((NBATCH, 1), jnp.float32),
)


def kernel(values, feat_idx, row_ids, weight):
    feat2d = feat_idx.reshape(NNZ // GB, GB)
    rids2d = row_ids.reshape(NNZ // GB, GB)
    part = _sc_kernel(values, feat2d, rids2d, weight)
    return _combine(part)


# speedup vs baseline: 1.0226x; 1.0226x over previous
import functools

import jax
import jax.numpy as jnp
from jax import lax
from jax.experimental import pallas as pl
from jax.experimental.pallas import tpu as pltpu
from jax.experimental.pallas import tpu_sc as plsc

NNZ = 1638400
VOCAB_SIZE = 1000000
KDIM = 16
NBATCH = 16384

NC = 2
NS = 16
NW = NC * NS
PER_W = NNZ // NW
CHUNK = 256
U = 4
NBODY = PER_W // (CHUNK * U)
GB = 128
NGB = CHUNK // GB
ROWS_PER_TILE = NBATCH // NS


def _sc_body(vals_hbm, feat2d_hbm, rids2d_hbm, weight_hbm, out_hbm,
             idx_v, rids_v, vals_v, rows_v, wbuf_v, zbuf_v, acc_sh,
             psem, gsem0, gsem1, gsem2, gsem3, ssem0, ssem1, ssem2, ssem3):
    c_id = lax.axis_index("c")
    s_id = lax.axis_index("s")
    wid = c_id * NS + s_id
    base = wid * PER_W

    z16 = jnp.zeros((16,), jnp.float32)
    z16i = jnp.zeros((16,), jnp.int32)
    iota16 = lax.broadcasted_iota(jnp.int32, (16,), 0)
    iota16h = iota16 + 16
    gsems = (gsem0, gsem1, gsem2, gsem3)
    ssems = (ssem0, ssem1, ssem2, ssem3)

    def zero_zbuf(r, carry):
        plsc.store_scatter(zbuf_v, [z16i + r, iota16], z16)
        plsc.store_scatter(zbuf_v, [z16i + r, iota16h], z16)
        return carry

    lax.fori_loop(0, 128, zero_zbuf, 0)

    def zero_acc(t, carry):
        pltpu.sync_copy(zbuf_v, acc_sh.at[pl.ds(pl.multiple_of(s_id * ROWS_PER_TILE + t * 128, 128), 128)])
        return carry

    lax.fori_loop(0, ROWS_PER_TILE // 128, zero_acc, 0)
    plsc.subcore_barrier()

    def issue_params(ci, u):
        row0 = pl.multiple_of(base // GB + ci * NGB, 8)
        cb = pl.multiple_of(base + ci * CHUNK, CHUNK)
        pltpu.async_copy(feat2d_hbm.at[pl.ds(row0, NGB)],
                         idx_v.at[pl.ds(u * NGB, NGB)], psem)
        pltpu.async_copy(rids2d_hbm.at[pl.ds(row0, NGB)],
                         rids_v.at[pl.ds(u * NGB, NGB)], psem)
        pltpu.async_copy(vals_hbm.at[pl.ds(cb, CHUNK)],
                         vals_v.at[pl.ds(u * CHUNK, CHUNK)], psem)

    def wait_params(u):
        pltpu.make_async_copy(feat2d_hbm.at[pl.ds(0, NGB)],
                              idx_v.at[pl.ds(u * NGB, NGB)], psem).wait()
        pltpu.make_async_copy(rids2d_hbm.at[pl.ds(0, NGB)],
                              rids_v.at[pl.ds(u * NGB, NGB)], psem).wait()
        pltpu.make_async_copy(vals_hbm.at[pl.ds(0, CHUNK)],
                              vals_v.at[pl.ds(u * CHUNK, CHUNK)], psem).wait()

    def compute(u):
        def group_body(g, carry2):
            gb = u * CHUNK + g * 16
            v16 = vals_v[pl.ds(gb, 16)]
            ridx = z16i + gb
            for l in range(16):
                row = plsc.load_gather(rows_v, [ridx + l, iota16])
                w = row * v16[l]
                plsc.store_scatter(wbuf_v, [ridx + l, iota16], w)
                plsc.store_scatter(wbuf_v, [ridx + l, iota16h], w * w)
            return carry2

        lax.fori_loop(0, CHUNK // 16, group_body, 0)

    BODY_NNZ = U * CHUNK
    BODY_ROWS = BODY_NNZ // GB

    def body(ii, carry):
        row0 = pl.multiple_of(base // GB + ii * BODY_ROWS, 8)
        cb = pl.multiple_of(base + ii * BODY_NNZ, BODY_NNZ)
        pltpu.sync_copy(feat2d_hbm.at[pl.ds(row0, BODY_ROWS)], idx_v)
        pltpu.sync_copy(rids2d_hbm.at[pl.ds(row0, BODY_ROWS)], rids_v)
        pltpu.sync_copy(vals_hbm.at[pl.ds(cb, BODY_NNZ)], vals_v)
        gds = [
            [pltpu.async_copy(weight_hbm.at[idx_v.at[u * NGB + j]],
                              rows_v.at[pl.ds(u * CHUNK + j * GB, GB)],
                              gsems[u])
             for j in range(NGB)]
            for u in range(U)
        ]
        for u in range(U):
            for d in gds[u]:
                d.wait()
            compute(u)
            for j in range(NGB):
                pltpu.sync_copy(wbuf_v.at[pl.ds(u * CHUNK + j * GB, GB)],
                                acc_sh.at[rids_v.at[u * NGB + j]], add=True)
        return carry

    lax.fori_loop(0, NBODY, body, 0)

    plsc.subcore_barrier()
    out_base = pl.multiple_of(s_id * ROWS_PER_TILE, ROWS_PER_TILE)
    pltpu.sync_copy(acc_sh.at[pl.ds(out_base, ROWS_PER_TILE)],
                    out_hbm.at[c_id, pl.ds(out_base, ROWS_PER_TILE)])


_sc_kernel = functools.partial(
    pl.kernel,
    mesh=plsc.VectorSubcoreMesh(core_axis_name="c", subcore_axis_name="s",
                                num_cores=NC, num_subcores=NS),
    out_type=jax.ShapeDtypeStruct((NC, NBATCH, 32), jnp.float32),
    scratch_types=[
        pltpu.VMEM((U * NGB, GB), jnp.int32),
        pltpu.VMEM((U * NGB, GB), jnp.int32),
        pltpu.VMEM((U * CHUNK,), jnp.float32),
        pltpu.VMEM((U * CHUNK, KDIM), jnp.float32),
        pltpu.VMEM((U * CHUNK, 32), jnp.float32),
        pltpu.VMEM((128, 32), jnp.float32),
        pltpu.VMEM_SHARED((NBATCH, 32), jnp.float32),
    ] + [pltpu.SemaphoreType.DMA] * 9,
    compiler_params=pltpu.CompilerParams(needs_layout_passes=False, use_tc_tiling_on_sc=False),
)(_sc_body)


def _combine_body(p_ref, o_ref):
    x = p_ref[...]
    p = x[0] + x[1]
    k = lax.broadcasted_iota(jnp.int32, (NBATCH, 32), 1)
    t = jnp.where(k < KDIM, p * p, -p)
    o_ref[...] = jnp.sum(t, axis=1, keepdims=True)


_combine = pl.pallas_call(
    _combine_body,
    out_shape=jax.ShapeDtypeStruct((NBATCH, 1), jnp.float32),
)


def kernel(values, feat_idx, row_ids, weight):
    feat2d = feat_idx.reshape(NNZ // GB, GB)
    rids2d = row_ids.reshape(NNZ // GB, GB)
    part = _sc_kernel(values, feat2d, rids2d, weight)
    return _combine(part)
